# baseline (speedup 1.0000x reference)
import functools

import jax
import jax.numpy as jnp
from jax import lax
from jax.experimental import pallas as pl
from jax.experimental.pallas import tpu as pltpu
from jax.experimental.pallas import tpu_sc as plsc

N = 10000
E = 320000
D = 128
NC, NS = 2, 16
NW = NC * NS
EPT = E // NW
CHUNK = 128
NCHUNK = 80
EPAD = NCHUNK * CHUNK - EPT
NPAD = 10240
ROWS_PER_TILE = NPAD // NS
GCH = 16
NGRP = NCHUNK // GCH


def _edge_body(compute_deg, x_hbm, src_hbm, dst_hbm, z2_hbm, z1_hbm,
               *refs):
  if compute_deg:
    (sum_out, deg_out, acc_sh, deg_sh, src_v, dst_v, rows_a, rows_b,
     ones_v, sem_a, sem_b) = refs
  else:
    (sum_out, acc_sh, src_v, dst_v, rows_a, rows_b, sem_a, sem_b) = refs

  c = lax.axis_index("c")
  s = lax.axis_index("s")
  wid = c * NS + s

  base = s * ROWS_PER_TILE
  pltpu.sync_copy(z2_hbm, acc_sh.at[pl.ds(base, ROWS_PER_TILE)])
  if compute_deg:
    pltpu.sync_copy(z1_hbm, deg_sh.at[pl.ds(base, ROWS_PER_TILE)])
    for k in range(CHUNK // 16):
      ones_v[pl.ds(k * 16, 16)] = jnp.full((16,), 1.0, jnp.float32)

  plsc.subcore_barrier()

  def start(j, buf, sem):
    return pltpu.async_copy(x_hbm.at[src_v.at[j]], buf, sem)

  def wait(buf, sem):
    pltpu.make_async_copy(x_hbm.at[src_v.at[0]], buf, sem).wait()

  def scatter(j, buf):
    pltpu.sync_copy(buf, acc_sh.at[dst_v.at[j]], add=True)
    if compute_deg:
      pltpu.sync_copy(ones_v, deg_sh.at[dst_v.at[j]], add=True)

  for g in range(NGRP):
    pltpu.sync_copy(src_hbm.at[wid, pl.ds(g * GCH, GCH)], src_v)
    pltpu.sync_copy(dst_hbm.at[wid, pl.ds(g * GCH, GCH)], dst_v)

    start(0, rows_a, sem_a)
    start(1, rows_b, sem_b)

    def body(i, carry):
      j = 2 * i
      wait(rows_a, sem_a)
      scatter(j, rows_a)
      start(j + 2, rows_a, sem_a)
      wait(rows_b, sem_b)
      scatter(j + 1, rows_b)
      start(j + 3, rows_b, sem_b)
      return carry

    lax.fori_loop(0, GCH // 2 - 1, body, 0, unroll=False)
    wait(rows_a, sem_a)
    scatter(GCH - 2, rows_a)
    wait(rows_b, sem_b)
    scatter(GCH - 1, rows_b)

  plsc.subcore_barrier()

  out_base = c * NPAD + base
  pltpu.sync_copy(acc_sh.at[pl.ds(base, ROWS_PER_TILE)],
                  sum_out.at[pl.ds(out_base, ROWS_PER_TILE)])
  if compute_deg:
    pltpu.sync_copy(deg_sh.at[pl.ds(base, ROWS_PER_TILE)],
                    deg_out.at[pl.ds(out_base, ROWS_PER_TILE)])


_edge_pass_deg = functools.partial(
    pl.kernel,
    out_type=(jax.ShapeDtypeStruct((NC * NPAD, D), jnp.float32),
              jax.ShapeDtypeStruct((NC * NPAD,), jnp.float32)),
    mesh=plsc.VectorSubcoreMesh(core_axis_name="c", subcore_axis_name="s"),
    scratch_types=(
        pltpu.VMEM_SHARED((NPAD, D), jnp.float32),
        pltpu.VMEM_SHARED((NPAD,), jnp.float32),
        pltpu.VMEM((GCH, CHUNK), jnp.int32),
        pltpu.VMEM((GCH, CHUNK), jnp.int32),
        pltpu.VMEM((CHUNK, D), jnp.float32),
        pltpu.VMEM((CHUNK, D), jnp.float32),
        pltpu.VMEM((CHUNK,), jnp.float32),
        pltpu.SemaphoreType.DMA,
        pltpu.SemaphoreType.DMA,
    ),
)(functools.partial(_edge_body, True))


_edge_pass_nodeg = functools.partial(
    pl.kernel,
    out_type=jax.ShapeDtypeStruct((NC * NPAD, D), jnp.float32),
    mesh=plsc.VectorSubcoreMesh(core_axis_name="c", subcore_axis_name="s"),
    scratch_types=(
        pltpu.VMEM_SHARED((NPAD, D), jnp.float32),
        pltpu.VMEM((GCH, CHUNK), jnp.int32),
        pltpu.VMEM((GCH, CHUNK), jnp.int32),
        pltpu.VMEM((CHUNK, D), jnp.float32),
        pltpu.VMEM((CHUNK, D), jnp.float32),
        pltpu.SemaphoreType.DMA,
        pltpu.SemaphoreType.DMA,
    ),
)(functools.partial(_edge_body, False))


_BLK = 1000
_row = lambda i: (i, 0)
_whole = lambda i: (0, 0)


def _dense_r_body(x, wr, b, o):
  o[:] = jnp.dot(x[:], wr[:], preferred_element_type=jnp.float32) + b[:]


def _dense_r(x, wr_t, b):
  return pl.pallas_call(
      _dense_r_body,
      grid=(N // _BLK,),
      in_specs=[
          pl.BlockSpec((_BLK, D), _row),
          pl.BlockSpec((D, D), _whole),
          pl.BlockSpec((1, D), _whole),
      ],
      out_specs=pl.BlockSpec((_BLK, D), _row),
      out_shape=jax.ShapeDtypeStruct((N, D), jnp.float32),
  )(x, wr_t, b)


def _dense_l_body(relu, s0, s1, d0, d1, r, wl, o):
  deg = jnp.maximum(d0[:] + d1[:], 1.0)
  mean = (s0[:] + s1[:]) / deg
  acc = jnp.dot(mean, wl[:], preferred_element_type=jnp.float32) + r[:]
  o[:] = jnp.maximum(acc, 0.0) if relu else acc


def _dense_l(s0, s1, d0, d1, r, wl_t, relu):
  return pl.pallas_call(
      functools.partial(_dense_l_body, relu),
      grid=(N // _BLK,),
      in_specs=[
          pl.BlockSpec((_BLK, D), _row),
          pl.BlockSpec((_BLK, D), _row),
          pl.BlockSpec((_BLK, 1), _row),
          pl.BlockSpec((_BLK, 1), _row),
          pl.BlockSpec((_BLK, D), _row),
          pl.BlockSpec((D, D), _whole),
      ],
      out_specs=pl.BlockSpec((_BLK, D), _row),
      out_shape=jax.ShapeDtypeStruct((N, D), jnp.float32),
  )(s0, s1, d0, d1, r, wl_t)


def _layer(x, src3, dst3, z2, z1, w_l, b_l, w_r, relu, degs=None):
  r = _dense_r(x, w_r.T, b_l.reshape(1, D))
  if degs is None:
    sums, degs = _edge_pass_deg(x, src3, dst3, z2, z1)
  else:
    sums = _edge_pass_nodeg(x, src3, dst3, z2, z1)
  s0 = sums[:N]
  s1 = sums[NPAD:NPAD + N]
  d0 = degs[:N].reshape(N, 1)
  d1 = degs[NPAD:NPAD + N].reshape(N, 1)
  out = _dense_l(s0, s1, d0, d1, r, w_l.T, relu)
  return out, degs


def kernel(x, edge_index, W_l1, b_l1, W_r1, W_l2, b_l2, W_r2):
  ei = edge_index.astype(jnp.int32)
  src2 = ei[0].reshape(NW, EPT)
  dst2 = ei[1].reshape(NW, EPT)
  pad_src = (jnp.arange(EPAD, dtype=jnp.int32)[None, :] * 37
             + 293 * jnp.arange(NW, dtype=jnp.int32)[:, None]) % N
  pad_dst = N + (jnp.arange(EPAD, dtype=jnp.int32)[None, :]
                 + jnp.arange(NW, dtype=jnp.int32)[:, None] * 7) % (NPAD - N)
  src3 = jnp.concatenate([src2, pad_src], axis=1).reshape(NW, NCHUNK, CHUNK)
  dst3 = jnp.concatenate([dst2, pad_dst], axis=1).reshape(NW, NCHUNK, CHUNK)
  z2 = jnp.zeros((ROWS_PER_TILE, D), jnp.float32)
  z1 = jnp.zeros((ROWS_PER_TILE,), jnp.float32)
  h, degs = _layer(x, src3, dst3, z2, z1, W_l1, b_l1, W_r1, relu=True)
  out, _ = _layer(h, src3, dst3, z2, z1, W_l2, b_l2, W_r2, relu=False,
                  degs=degs)
  return out


# speedup vs baseline: 1.0350x; 1.0350x over previous
import functools

import jax
import jax.numpy as jnp
from jax import lax
from jax.experimental import pallas as pl
from jax.experimental.pallas import tpu as pltpu
from jax.experimental.pallas import tpu_sc as plsc

N = 10000
E = 320000
D = 128
NC, NS = 2, 16
NW = NC * NS
EPT = E // NW
CHUNK = 72
NCHUNK = 144
EPAD = NCHUNK * CHUNK - EPT
NPAD = 10240
ROWS_PER_TILE = NPAD // NS
GCH = 24
NGRP = NCHUNK // GCH
NBUF = 4


def _edge_body(compute_deg, x_hbm, src_hbm, dst_hbm, z2_hbm, z1_hbm,
               *refs):
  if compute_deg:
    (sum_out, deg_out, acc_sh, deg_sh, src_v, dst_v,
     r0, r1, r2, r3, ones_v, s0, s1, s2, s3) = refs
  else:
    (sum_out, acc_sh, src_v, dst_v, r0, r1, r2, r3, s0, s1, s2, s3) = refs
  bufs = (r0, r1, r2, r3)
  sems = (s0, s1, s2, s3)

  c = lax.axis_index("c")
  s = lax.axis_index("s")
  wid = c * NS + s

  base = s * ROWS_PER_TILE
  pltpu.sync_copy(z2_hbm, acc_sh.at[pl.ds(base, ROWS_PER_TILE)])
  if compute_deg:
    pltpu.sync_copy(z1_hbm, deg_sh.at[pl.ds(base, ROWS_PER_TILE)])
    for k in range(CHUNK // 16):
      ones_v[pl.ds(k * 16, 16)] = jnp.full((16,), 1.0, jnp.float32)

  plsc.subcore_barrier()

  def start(j, buf, sem):
    return pltpu.async_copy(x_hbm.at[src_v.at[j]], buf, sem)

  def wait(buf, sem):
    pltpu.make_async_copy(x_hbm.at[src_v.at[0]], buf, sem).wait()

  def scatter(j, buf):
    pltpu.sync_copy(buf, acc_sh.at[dst_v.at[j]], add=True)
    if compute_deg:
      pltpu.sync_copy(ones_v, deg_sh.at[dst_v.at[j]], add=True)

  for g in range(NGRP):
    pltpu.sync_copy(src_hbm.at[wid, pl.ds(g * GCH, GCH)], src_v)
    pltpu.sync_copy(dst_hbm.at[wid, pl.ds(g * GCH, GCH)], dst_v)

    for k in range(NBUF):
      start(k, bufs[k], sems[k])

    def body(i, carry):
      j = NBUF * i
      for k in range(NBUF):
        wait(bufs[k], sems[k])
        scatter(j + k, bufs[k])
        start(j + NBUF + k, bufs[k], sems[k])
      return carry

    lax.fori_loop(0, GCH // NBUF - 1, body, 0, unroll=False)
    for k in range(NBUF):
      wait(bufs[k], sems[k])
      scatter(GCH - NBUF + k, bufs[k])

  plsc.subcore_barrier()

  out_base = c * NPAD + base
  pltpu.sync_copy(acc_sh.at[pl.ds(base, ROWS_PER_TILE)],
                  sum_out.at[pl.ds(out_base, ROWS_PER_TILE)])
  if compute_deg:
    pltpu.sync_copy(deg_sh.at[pl.ds(base, ROWS_PER_TILE)],
                    deg_out.at[pl.ds(out_base, ROWS_PER_TILE)])


_edge_pass_deg = functools.partial(
    pl.kernel,
    out_type=(jax.ShapeDtypeStruct((NC * NPAD, D), jnp.float32),
              jax.ShapeDtypeStruct((NC * NPAD,), jnp.float32)),
    mesh=plsc.VectorSubcoreMesh(core_axis_name="c", subcore_axis_name="s"),
    scratch_types=(
        pltpu.VMEM_SHARED((NPAD, D), jnp.float32),
        pltpu.VMEM_SHARED((NPAD,), jnp.float32),
        pltpu.VMEM((GCH, CHUNK), jnp.int32),
        pltpu.VMEM((GCH, CHUNK), jnp.int32),
        pltpu.VMEM((CHUNK, D), jnp.float32),
        pltpu.VMEM((CHUNK, D), jnp.float32),
        pltpu.VMEM((CHUNK, D), jnp.float32),
        pltpu.VMEM((CHUNK, D), jnp.float32),
        pltpu.VMEM((CHUNK,), jnp.float32),
        pltpu.SemaphoreType.DMA,
        pltpu.SemaphoreType.DMA,
        pltpu.SemaphoreType.DMA,
        pltpu.SemaphoreType.DMA,
    ),
)(functools.partial(_edge_body, True))


_edge_pass_nodeg = functools.partial(
    pl.kernel,
    out_type=jax.ShapeDtypeStruct((NC * NPAD, D), jnp.float32),
    mesh=plsc.VectorSubcoreMesh(core_axis_name="c", subcore_axis_name="s"),
    scratch_types=(
        pltpu.VMEM_SHARED((NPAD, D), jnp.float32),
        pltpu.VMEM((GCH, CHUNK), jnp.int32),
        pltpu.VMEM((GCH, CHUNK), jnp.int32),
        pltpu.VMEM((CHUNK, D), jnp.float32),
        pltpu.VMEM((CHUNK, D), jnp.float32),
        pltpu.VMEM((CHUNK, D), jnp.float32),
        pltpu.VMEM((CHUNK, D), jnp.float32),
        pltpu.SemaphoreType.DMA,
        pltpu.SemaphoreType.DMA,
        pltpu.SemaphoreType.DMA,
        pltpu.SemaphoreType.DMA,
    ),
)(functools.partial(_edge_body, False))


def _dense_body(relu, s0, s1, d0, d1, x, wl, wr, b, o):
  deg = jnp.maximum(d0[:] + d1[:], 1.0)
  mean = (s0[:] + s1[:]) / deg
  acc = (jnp.dot(mean, wl[:], preferred_element_type=jnp.float32)
         + jnp.dot(x[:], wr[:], preferred_element_type=jnp.float32)
         + b[:])
  o[:] = jnp.maximum(acc, 0.0) if relu else acc


def _dense(s0, s1, d0, d1, x, wl_t, wr_t, b, relu):
  blk = 1000
  grid = N // blk
  row = lambda i: (i, 0)
  whole = lambda i: (0, 0)
  return pl.pallas_call(
      functools.partial(_dense_body, relu),
      grid=(grid,),
      in_specs=[
          pl.BlockSpec((blk, D), row),
          pl.BlockSpec((blk, D), row),
          pl.BlockSpec((blk, 1), row),
          pl.BlockSpec((blk, 1), row),
          pl.BlockSpec((blk, D), row),
          pl.BlockSpec((D, D), whole),
          pl.BlockSpec((D, D), whole),
          pl.BlockSpec((1, D), whole),
      ],
      out_specs=pl.BlockSpec((blk, D), row),
      out_shape=jax.ShapeDtypeStruct((N, D), jnp.float32),
  )(s0, s1, d0, d1, x, wl_t, wr_t, b)


def _layer(x, src3, dst3, z2, z1, w_l, b_l, w_r, relu, degs=None):
  if degs is None:
    sums, degs = _edge_pass_deg(x, src3, dst3, z2, z1)
  else:
    sums = _edge_pass_nodeg(x, src3, dst3, z2, z1)
  s0 = sums[:N]
  s1 = sums[NPAD:NPAD + N]
  d0 = degs[:N].reshape(N, 1)
  d1 = degs[NPAD:NPAD + N].reshape(N, 1)
  out = _dense(s0, s1, d0, d1, x, w_l.T, w_r.T, b_l.reshape(1, D), relu)
  return out, degs


def kernel(x, edge_index, W_l1, b_l1, W_r1, W_l2, b_l2, W_r2):
  ei = edge_index.astype(jnp.int32)
  src2 = ei[0].reshape(NW, EPT)
  dst2 = ei[1].reshape(NW, EPT)
  pad_src = (jnp.arange(EPAD, dtype=jnp.int32)[None, :] * 37
             + 293 * jnp.arange(NW, dtype=jnp.int32)[:, None]) % N
  pad_dst = N + (jnp.arange(EPAD, dtype=jnp.int32)[None, :]
                 + jnp.arange(NW, dtype=jnp.int32)[:, None] * 7) % (NPAD - N)
  src3 = jnp.concatenate([src2, pad_src], axis=1).reshape(NW, NCHUNK, CHUNK)
  dst3 = jnp.concatenate([dst2, pad_dst], axis=1).reshape(NW, NCHUNK, CHUNK)
  z2 = jnp.zeros((ROWS_PER_TILE, D), jnp.float32)
  z1 = jnp.zeros((ROWS_PER_TILE,), jnp.float32)
  h, degs = _layer(x, src3, dst3, z2, z1, W_l1, b_l1, W_r1, relu=True)
  out, _ = _layer(h, src3, dst3, z2, z1, W_l2, b_l2, W_r2, relu=False,
                  degs=degs)
  return out


# speedup vs baseline: 1.0643x; 1.0282x over previous
import functools

import jax
import jax.numpy as jnp
from jax import lax
from jax.experimental import pallas as pl
from jax.experimental.pallas import tpu as pltpu
from jax.experimental.pallas import tpu_sc as plsc

N = 10000
E = 320000
D = 128
NC, NS = 2, 16
NW = NC * NS
EPT = E // NW
CHUNK = 64
NCHUNK = 160
EPAD = NCHUNK * CHUNK - EPT
NPAD = 10240
ROWS_PER_TILE = NPAD // NS
GCH = 32
NGRP = NCHUNK // GCH
NBUF = 4


def _edge_body(compute_deg, x_hbm, src_hbm, dst_hbm, z2_hbm, z1_hbm,
               *refs):
  if compute_deg:
    (sum_out, deg_out, acc_sh, deg_sh, src_v, dst_v,
     r0, r1, r2, r3, ones_v, s0, s1, s2, s3) = refs
  else:
    (sum_out, acc_sh, src_v, dst_v, r0, r1, r2, r3, s0, s1, s2, s3) = refs
  bufs = (r0, r1, r2, r3)
  sems = (s0, s1, s2, s3)

  c = lax.axis_index("c")
  s = lax.axis_index("s")
  wid = c * NS + s

  base = s * ROWS_PER_TILE
  pltpu.sync_copy(z2_hbm, acc_sh.at[pl.ds(base, ROWS_PER_TILE)])
  if compute_deg:
    pltpu.sync_copy(z1_hbm, deg_sh.at[pl.ds(base, ROWS_PER_TILE)])
    for k in range(CHUNK // 16):
      ones_v[pl.ds(k * 16, 16)] = jnp.full((16,), 1.0, jnp.float32)

  plsc.subcore_barrier()

  def start(j, buf, sem):
    return pltpu.async_copy(x_hbm.at[src_v.at[j]], buf, sem)

  def wait(buf, sem):
    pltpu.make_async_copy(x_hbm.at[src_v.at[0]], buf, sem).wait()

  def scatter(j, buf):
    pltpu.sync_copy(buf, acc_sh.at[dst_v.at[j]], add=True)
    if compute_deg:
      pltpu.sync_copy(ones_v, deg_sh.at[dst_v.at[j]], add=True)

  for g in range(NGRP):
    pltpu.sync_copy(src_hbm.at[wid, pl.ds(g * GCH, GCH)], src_v)
    pltpu.sync_copy(dst_hbm.at[wid, pl.ds(g * GCH, GCH)], dst_v)

    for k in range(NBUF):
      start(k, bufs[k], sems[k])

    def body(i, carry):
      j = NBUF * i
      for k in range(NBUF):
        wait(bufs[k], sems[k])
        scatter(j + k, bufs[k])
        start(j + NBUF + k, bufs[k], sems[k])
      return carry

    lax.fori_loop(0, GCH // NBUF - 1, body, 0, unroll=False)
    for k in range(NBUF):
      wait(bufs[k], sems[k])
      scatter(GCH - NBUF + k, bufs[k])

  plsc.subcore_barrier()

  out_base = c * NPAD + base
  pltpu.sync_copy(acc_sh.at[pl.ds(base, ROWS_PER_TILE)],
                  sum_out.at[pl.ds(out_base, ROWS_PER_TILE)])
  if compute_deg:
    pltpu.sync_copy(deg_sh.at[pl.ds(base, ROWS_PER_TILE)],
                    deg_out.at[pl.ds(out_base, ROWS_PER_TILE)])


_edge_pass_deg = functools.partial(
    pl.kernel,
    out_type=(jax.ShapeDtypeStruct((NC * NPAD, D), jnp.float32),
              jax.ShapeDtypeStruct((NC * NPAD,), jnp.float32)),
    mesh=plsc.VectorSubcoreMesh(core_axis_name="c", subcore_axis_name="s"),
    scratch_types=(
        pltpu.VMEM_SHARED((NPAD, D), jnp.float32),
        pltpu.VMEM_SHARED((NPAD,), jnp.float32),
        pltpu.VMEM((GCH, CHUNK), jnp.int32),
        pltpu.VMEM((GCH, CHUNK), jnp.int32),
        pltpu.VMEM((CHUNK, D), jnp.float32),
        pltpu.VMEM((CHUNK, D), jnp.float32),
        pltpu.VMEM((CHUNK, D), jnp.float32),
        pltpu.VMEM((CHUNK, D), jnp.float32),
        pltpu.VMEM((CHUNK,), jnp.float32),
        pltpu.SemaphoreType.DMA,
        pltpu.SemaphoreType.DMA,
        pltpu.SemaphoreType.DMA,
        pltpu.SemaphoreType.DMA,
    ),
)(functools.partial(_edge_body, True))


_edge_pass_nodeg = functools.partial(
    pl.kernel,
    out_type=jax.ShapeDtypeStruct((NC * NPAD, D), jnp.float32),
    mesh=plsc.VectorSubcoreMesh(core_axis_name="c", subcore_axis_name="s"),
    scratch_types=(
        pltpu.VMEM_SHARED((NPAD, D), jnp.float32),
        pltpu.VMEM((GCH, CHUNK), jnp.int32),
        pltpu.VMEM((GCH, CHUNK), jnp.int32),
        pltpu.VMEM((CHUNK, D), jnp.float32),
        pltpu.VMEM((CHUNK, D), jnp.float32),
        pltpu.VMEM((CHUNK, D), jnp.float32),
        pltpu.VMEM((CHUNK, D), jnp.float32),
        pltpu.SemaphoreType.DMA,
        pltpu.SemaphoreType.DMA,
        pltpu.SemaphoreType.DMA,
        pltpu.SemaphoreType.DMA,
    ),
)(functools.partial(_edge_body, False))


def _dense_body(relu, s0, s1, d0, d1, x, wl, wr, b, o):
  deg = jnp.maximum(d0[:] + d1[:], 1.0)
  mean = (s0[:] + s1[:]) / deg
  acc = (jnp.dot(mean, wl[:], preferred_element_type=jnp.float32)
         + jnp.dot(x[:], wr[:], preferred_element_type=jnp.float32)
         + b[:])
  o[:] = jnp.maximum(acc, 0.0) if relu else acc


def _dense(s0, s1, d0, d1, x, wl_t, wr_t, b, relu):
  blk = 1000
  grid = N // blk
  row = lambda i: (i, 0)
  whole = lambda i: (0, 0)
  return pl.pallas_call(
      functools.partial(_dense_body, relu),
      grid=(grid,),
      in_specs=[
          pl.BlockSpec((blk, D), row),
          pl.BlockSpec((blk, D), row),
          pl.BlockSpec((blk, 1), row),
          pl.BlockSpec((blk, 1), row),
          pl.BlockSpec((blk, D), row),
          pl.BlockSpec((D, D), whole),
          pl.BlockSpec((D, D), whole),
          pl.BlockSpec((1, D), whole),
      ],
      out_specs=pl.BlockSpec((blk, D), row),
      out_shape=jax.ShapeDtypeStruct((N, D), jnp.float32),
  )(s0, s1, d0, d1, x, wl_t, wr_t, b)


def _layer(x, src3, dst3, z2, z1, w_l, b_l, w_r, relu, degs=None):
  if degs is None:
    sums, degs = _edge_pass_deg(x, src3, dst3, z2, z1)
  else:
    sums = _edge_pass_nodeg(x, src3, dst3, z2, z1)
  s0 = sums[:N]
  s1 = sums[NPAD:NPAD + N]
  d0 = degs[:N].reshape(N, 1)
  d1 = degs[NPAD:NPAD + N].reshape(N, 1)
  out = _dense(s0, s1, d0, d1, x, w_l.T, w_r.T, b_l.reshape(1, D), relu)
  return out, degs


def kernel(x, edge_index, W_l1, b_l1, W_r1, W_l2, b_l2, W_r2):
  ei = edge_index.astype(jnp.int32)
  src2 = ei[0].reshape(NW, EPT)
  dst2 = ei[1].reshape(NW, EPT)
  pad_src = (jnp.arange(EPAD, dtype=jnp.int32)[None, :] * 37
             + 293 * jnp.arange(NW, dtype=jnp.int32)[:, None]) % N
  pad_dst = N + (jnp.arange(EPAD, dtype=jnp.int32)[None, :]
                 + jnp.arange(NW, dtype=jnp.int32)[:, None] * 7) % (NPAD - N)
  src3 = jnp.concatenate([src2, pad_src], axis=1).reshape(NW, NCHUNK, CHUNK)
  dst3 = jnp.concatenate([dst2, pad_dst], axis=1).reshape(NW, NCHUNK, CHUNK)
  z2 = jnp.zeros((ROWS_PER_TILE, D), jnp.float32)
  z1 = jnp.zeros((ROWS_PER_TILE,), jnp.float32)
  h, degs = _layer(x, src3, dst3, z2, z1, W_l1, b_l1, W_r1, relu=True)
  out, _ = _layer(h, src3, dst3, z2, z1, W_l2, b_l2, W_r2, relu=False,
                  degs=degs)
  return out


# speedup vs baseline: 1.0986x; 1.0322x over previous
import functools

import jax
import jax.numpy as jnp
from jax import lax
from jax.experimental import pallas as pl
from jax.experimental.pallas import tpu as pltpu
from jax.experimental.pallas import tpu_sc as plsc

N = 10000
E = 320000
D = 128
NC, NS = 2, 16
NW = NC * NS
EPT = E // NW
CHUNK = 80
NCHUNK = 128
EPAD = NCHUNK * CHUNK - EPT
NPAD = 10112
ROWS_PER_TILE = NPAD // NS
DPAD = 10240
DROWS = DPAD // NS
GCH = 32
NGRP = NCHUNK // GCH
NBUF = 4


def _edge_body(compute_deg, x_hbm, src_hbm, dst_hbm, z2_hbm, z1_hbm,
               *refs):
  if compute_deg:
    (sum_out, deg_out, acc_sh, deg_sh, src_v, dst_v,
     r0, r1, r2, r3, ones_v, s0, s1, s2, s3) = refs
  else:
    (sum_out, acc_sh, src_v, dst_v, r0, r1, r2, r3, s0, s1, s2, s3) = refs
  bufs = (r0, r1, r2, r3)
  sems = (s0, s1, s2, s3)

  c = lax.axis_index("c")
  s = lax.axis_index("s")
  wid = c * NS + s

  base = s * ROWS_PER_TILE
  pltpu.sync_copy(z2_hbm, acc_sh.at[pl.ds(base, ROWS_PER_TILE)])
  if compute_deg:
    pltpu.sync_copy(z1_hbm, deg_sh.at[pl.ds(s * DROWS, DROWS)])
    for k in range(CHUNK // 16):
      ones_v[pl.ds(k * 16, 16)] = jnp.full((16,), 1.0, jnp.float32)

  plsc.subcore_barrier()

  def start(j, buf, sem):
    return pltpu.async_copy(x_hbm.at[src_v.at[j]], buf, sem)

  def wait(buf, sem):
    pltpu.make_async_copy(x_hbm.at[src_v.at[0]], buf, sem).wait()

  def scatter(j, buf):
    pltpu.sync_copy(buf, acc_sh.at[dst_v.at[j]], add=True)
    if compute_deg:
      pltpu.sync_copy(ones_v, deg_sh.at[dst_v.at[j]], add=True)

  for g in range(NGRP):
    pltpu.sync_copy(src_hbm.at[wid, pl.ds(g * GCH, GCH)], src_v)
    pltpu.sync_copy(dst_hbm.at[wid, pl.ds(g * GCH, GCH)], dst_v)

    for k in range(NBUF):
      start(k, bufs[k], sems[k])

    def body(i, carry):
      j = NBUF * i
      for k in range(NBUF):
        wait(bufs[k], sems[k])
        scatter(j + k, bufs[k])
        start(j + NBUF + k, bufs[k], sems[k])
      return carry

    lax.fori_loop(0, GCH // NBUF - 1, body, 0, unroll=False)
    for k in range(NBUF):
      wait(bufs[k], sems[k])
      scatter(GCH - NBUF + k, bufs[k])

  plsc.subcore_barrier()

  out_base = c * NPAD + base
  pltpu.sync_copy(acc_sh.at[pl.ds(base, ROWS_PER_TILE)],
                  sum_out.at[pl.ds(out_base, ROWS_PER_TILE)])
  if compute_deg:
    pltpu.sync_copy(deg_sh.at[pl.ds(s * DROWS, DROWS)],
                    deg_out.at[pl.ds(c * DPAD + s * DROWS, DROWS)])


_edge_pass_deg = functools.partial(
    pl.kernel,
    out_type=(jax.ShapeDtypeStruct((NC * NPAD, D), jnp.float32),
              jax.ShapeDtypeStruct((NC * DPAD,), jnp.float32)),
    mesh=plsc.VectorSubcoreMesh(core_axis_name="c", subcore_axis_name="s"),
    scratch_types=(
        pltpu.VMEM_SHARED((NPAD, D), jnp.float32),
        pltpu.VMEM_SHARED((DPAD,), jnp.float32),
        pltpu.VMEM((GCH, CHUNK), jnp.int32),
        pltpu.VMEM((GCH, CHUNK), jnp.int32),
        pltpu.VMEM((CHUNK, D), jnp.float32),
        pltpu.VMEM((CHUNK, D), jnp.float32),
        pltpu.VMEM((CHUNK, D), jnp.float32),
        pltpu.VMEM((CHUNK, D), jnp.float32),
        pltpu.VMEM((CHUNK,), jnp.float32),
        pltpu.SemaphoreType.DMA,
        pltpu.SemaphoreType.DMA,
        pltpu.SemaphoreType.DMA,
        pltpu.SemaphoreType.DMA,
    ),
)(functools.partial(_edge_body, True))


_edge_pass_nodeg = functools.partial(
    pl.kernel,
    out_type=jax.ShapeDtypeStruct((NC * NPAD, D), jnp.float32),
    mesh=plsc.VectorSubcoreMesh(core_axis_name="c", subcore_axis_name="s"),
    scratch_types=(
        pltpu.VMEM_SHARED((NPAD, D), jnp.float32),
        pltpu.VMEM((GCH, CHUNK), jnp.int32),
        pltpu.VMEM((GCH, CHUNK), jnp.int32),
        pltpu.VMEM((CHUNK, D), jnp.float32),
        pltpu.VMEM((CHUNK, D), jnp.float32),
        pltpu.VMEM((CHUNK, D), jnp.float32),
        pltpu.VMEM((CHUNK, D), jnp.float32),
        pltpu.SemaphoreType.DMA,
        pltpu.SemaphoreType.DMA,
        pltpu.SemaphoreType.DMA,
        pltpu.SemaphoreType.DMA,
    ),
)(functools.partial(_edge_body, False))


def _dense_body(relu, s0, s1, d0, d1, x, wl, wr, b, o):
  deg = jnp.maximum(d0[:] + d1[:], 1.0)
  mean = (s0[:] + s1[:]) / deg
  acc = (jnp.dot(mean, wl[:], preferred_element_type=jnp.float32)
         + jnp.dot(x[:], wr[:], preferred_element_type=jnp.float32)
         + b[:])
  o[:] = jnp.maximum(acc, 0.0) if relu else acc


def _dense(s0, s1, d0, d1, x, wl_t, wr_t, b, relu):
  blk = 1000
  grid = N // blk
  row = lambda i: (i, 0)
  whole = lambda i: (0, 0)
  return pl.pallas_call(
      functools.partial(_dense_body, relu),
      grid=(grid,),
      in_specs=[
          pl.BlockSpec((blk, D), row),
          pl.BlockSpec((blk, D), row),
          pl.BlockSpec((blk, 1), row),
          pl.BlockSpec((blk, 1), row),
          pl.BlockSpec((blk, D), row),
          pl.BlockSpec((D, D), whole),
          pl.BlockSpec((D, D), whole),
          pl.BlockSpec((1, D), whole),
      ],
      out_specs=pl.BlockSpec((blk, D), row),
      out_shape=jax.ShapeDtypeStruct((N, D), jnp.float32),
  )(s0, s1, d0, d1, x, wl_t, wr_t, b)


def _layer(x, src3, dst3, z2, z1, w_l, b_l, w_r, relu, degs=None):
  if degs is None:
    sums, degs = _edge_pass_deg(x, src3, dst3, z2, z1)
  else:
    sums = _edge_pass_nodeg(x, src3, dst3, z2, z1)
  s0 = sums[:N]
  s1 = sums[NPAD:NPAD + N]
  d0 = degs[:N].reshape(N, 1)
  d1 = degs[DPAD:DPAD + N].reshape(N, 1)
  out = _dense(s0, s1, d0, d1, x, w_l.T, w_r.T, b_l.reshape(1, D), relu)
  return out, degs


def kernel(x, edge_index, W_l1, b_l1, W_r1, W_l2, b_l2, W_r2):
  ei = edge_index.astype(jnp.int32)
  src2 = ei[0].reshape(NW, EPT)
  dst2 = ei[1].reshape(NW, EPT)
  pad_src = (jnp.arange(EPAD, dtype=jnp.int32)[None, :] * 37
             + 293 * jnp.arange(NW, dtype=jnp.int32)[:, None]) % N
  pad_dst = N + (jnp.arange(EPAD, dtype=jnp.int32)[None, :]
                 + jnp.arange(NW, dtype=jnp.int32)[:, None] * 7) % (NPAD - N)
  src3 = jnp.concatenate([src2, pad_src], axis=1).reshape(NW, NCHUNK, CHUNK)
  dst3 = jnp.concatenate([dst2, pad_dst], axis=1).reshape(NW, NCHUNK, CHUNK)
  z2 = jnp.zeros((ROWS_PER_TILE, D), jnp.float32)
  z1 = jnp.zeros((DROWS,), jnp.float32)
  h, degs = _layer(x, src3, dst3, z2, z1, W_l1, b_l1, W_r1, relu=True)
  out, _ = _layer(h, src3, dst3, z2, z1, W_l2, b_l2, W_r2, relu=False,
                  degs=degs)
  return out


# speedup vs baseline: 1.1624x; 1.0581x over previous
import functools

import jax
import jax.numpy as jnp
from jax import lax
from jax.experimental import pallas as pl
from jax.experimental.pallas import tpu as pltpu
from jax.experimental.pallas import tpu_sc as plsc

N = 10000
E = 320000
D = 128
NC, NS = 2, 16
NW = NC * NS
EPT = E // NW
CHUNK = 64
NCHUNK = 160
EPAD = NCHUNK * CHUNK - EPT
NPAD = 10112
ROWS_PER_TILE = NPAD // NS
DPAD = 10240
DROWS = DPAD // NS
GCH = 32
NGRP = NCHUNK // GCH
NBUF = 4


def _edge_body(compute_deg, x_hbm, src_hbm, dst_hbm, z2_hbm, z1_hbm,
               *refs):
  if compute_deg:
    (sum_out, deg_out, acc_sh, deg_sh, src_a, dst_a, src_b, dst_b,
     r0, r1, r2, r3, ones_v, s0, s1, s2, s3, sem_i) = refs
  else:
    (sum_out, acc_sh, src_a, dst_a, src_b, dst_b,
     r0, r1, r2, r3, s0, s1, s2, s3, sem_i) = refs
  bufs = (r0, r1, r2, r3)
  sems = (s0, s1, s2, s3)
  idx_sets = ((src_a, dst_a), (src_b, dst_b))

  c = lax.axis_index("c")
  s = lax.axis_index("s")
  wid = c * NS + s

  base = s * ROWS_PER_TILE
  pltpu.sync_copy(z2_hbm, acc_sh.at[pl.ds(base, ROWS_PER_TILE)])
  if compute_deg:
    pltpu.sync_copy(z1_hbm, deg_sh.at[pl.ds(s * DROWS, DROWS)])
    for k in range(CHUNK // 16):
      ones_v[pl.ds(k * 16, 16)] = jnp.full((16,), 1.0, jnp.float32)

  plsc.subcore_barrier()

  def start(src_v, j, buf, sem):
    return pltpu.async_copy(x_hbm.at[src_v.at[j]], buf, sem)

  def wait(buf, sem):
    pltpu.make_async_copy(x_hbm.at[src_a.at[0]], buf, sem).wait()

  def scatter(dst_v, j, buf):
    pltpu.sync_copy(buf, acc_sh.at[dst_v.at[j]], add=True)
    if compute_deg:
      pltpu.sync_copy(ones_v, deg_sh.at[dst_v.at[j]], add=True)

  def stage(g, sbuf, dbuf):
    c1 = pltpu.async_copy(src_hbm.at[wid, pl.ds(g * GCH, GCH)], sbuf, sem_i)
    c2 = pltpu.async_copy(dst_hbm.at[wid, pl.ds(g * GCH, GCH)], dbuf, sem_i)
    return (c1, c2)

  pend = stage(0, src_a, dst_a)
  pend[0].wait()
  pend[1].wait()
  pend = stage(1, src_b, dst_b) if NGRP > 1 else None

  for k in range(NBUF):
    start(src_a, k, bufs[k], sems[k])

  for g in range(NGRP):
    cur_src, cur_dst = idx_sets[g % 2]
    nxt = idx_sets[(g + 1) % 2]

    def body(i, carry):
      j = NBUF * i
      for k in range(NBUF):
        wait(bufs[k], sems[k])
        scatter(cur_dst, j + k, bufs[k])
        start(cur_src, j + NBUF + k, bufs[k], sems[k])
      return carry

    lax.fori_loop(0, GCH // NBUF - 1, body, 0, unroll=False)

    if g + 1 < NGRP:
      pend[0].wait()
      pend[1].wait()
      for k in range(NBUF):
        wait(bufs[k], sems[k])
        scatter(cur_dst, GCH - NBUF + k, bufs[k])
        start(nxt[0], k, bufs[k], sems[k])
      if g + 2 < NGRP:
        pend = stage(g + 2, cur_src, cur_dst)
    else:
      for k in range(NBUF):
        wait(bufs[k], sems[k])
        scatter(cur_dst, GCH - NBUF + k, bufs[k])

  plsc.subcore_barrier()

  out_base = c * NPAD + base
  pltpu.sync_copy(acc_sh.at[pl.ds(base, ROWS_PER_TILE)],
                  sum_out.at[pl.ds(out_base, ROWS_PER_TILE)])
  if compute_deg:
    pltpu.sync_copy(deg_sh.at[pl.ds(s * DROWS, DROWS)],
                    deg_out.at[pl.ds(c * DPAD + s * DROWS, DROWS)])


_edge_pass_deg = functools.partial(
    pl.kernel,
    out_type=(jax.ShapeDtypeStruct((NC * NPAD, D), jnp.float32),
              jax.ShapeDtypeStruct((NC * DPAD,), jnp.float32)),
    mesh=plsc.VectorSubcoreMesh(core_axis_name="c", subcore_axis_name="s"),
    scratch_types=(
        pltpu.VMEM_SHARED((NPAD, D), jnp.float32),
        pltpu.VMEM_SHARED((DPAD,), jnp.float32),
        pltpu.VMEM((GCH, CHUNK), jnp.int32),
        pltpu.VMEM((GCH, CHUNK), jnp.int32),
        pltpu.VMEM((GCH, CHUNK), jnp.int32),
        pltpu.VMEM((GCH, CHUNK), jnp.int32),
        pltpu.VMEM((CHUNK, D), jnp.float32),
        pltpu.VMEM((CHUNK, D), jnp.float32),
        pltpu.VMEM((CHUNK, D), jnp.float32),
        pltpu.VMEM((CHUNK, D), jnp.float32),
        pltpu.VMEM((CHUNK,), jnp.float32),
        pltpu.SemaphoreType.DMA,
        pltpu.SemaphoreType.DMA,
        pltpu.SemaphoreType.DMA,
        pltpu.SemaphoreType.DMA,
        pltpu.SemaphoreType.DMA,
    ),
)(functools.partial(_edge_body, True))


_edge_pass_nodeg = functools.partial(
    pl.kernel,
    out_type=jax.ShapeDtypeStruct((NC * NPAD, D), jnp.float32),
    mesh=plsc.VectorSubcoreMesh(core_axis_name="c", subcore_axis_name="s"),
    scratch_types=(
        pltpu.VMEM_SHARED((NPAD, D), jnp.float32),
        pltpu.VMEM((GCH, CHUNK), jnp.int32),
        pltpu.VMEM((GCH, CHUNK), jnp.int32),
        pltpu.VMEM((GCH, CHUNK), jnp.int32),
        pltpu.VMEM((GCH, CHUNK), jnp.int32),
        pltpu.VMEM((CHUNK, D), jnp.float32),
        pltpu.VMEM((CHUNK, D), jnp.float32),
        pltpu.VMEM((CHUNK, D), jnp.float32),
        pltpu.VMEM((CHUNK, D), jnp.float32),
        pltpu.SemaphoreType.DMA,
        pltpu.SemaphoreType.DMA,
        pltpu.SemaphoreType.DMA,
        pltpu.SemaphoreType.DMA,
        pltpu.SemaphoreType.DMA,
    ),
)(functools.partial(_edge_body, False))


def _dense_body(relu, s0, s1, d0, d1, x, wl, wr, b, o):
  deg = jnp.maximum(d0[:] + d1[:], 1.0)
  mean = (s0[:] + s1[:]) / deg
  acc = (jnp.dot(mean, wl[:], preferred_element_type=jnp.float32)
         + jnp.dot(x[:], wr[:], preferred_element_type=jnp.float32)
         + b[:])
  o[:] = jnp.maximum(acc, 0.0) if relu else acc


def _dense(s0, s1, d0, d1, x, wl_t, wr_t, b, relu):
  blk = 1000
  grid = N // blk
  row = lambda i: (i, 0)
  whole = lambda i: (0, 0)
  return pl.pallas_call(
      functools.partial(_dense_body, relu),
      grid=(grid,),
      in_specs=[
          pl.BlockSpec((blk, D), row),
          pl.BlockSpec((blk, D), row),
          pl.BlockSpec((blk, 1), row),
          pl.BlockSpec((blk, 1), row),
          pl.BlockSpec((blk, D), row),
          pl.BlockSpec((D, D), whole),
          pl.BlockSpec((D, D), whole),
          pl.BlockSpec((1, D), whole),
      ],
      out_specs=pl.BlockSpec((blk, D), row),
      out_shape=jax.ShapeDtypeStruct((N, D), jnp.float32),
  )(s0, s1, d0, d1, x, wl_t, wr_t, b)


def _layer(x, src3, dst3, z2, z1, w_l, b_l, w_r, relu, degs=None):
  if degs is None:
    sums, degs = _edge_pass_deg(x, src3, dst3, z2, z1)
  else:
    sums = _edge_pass_nodeg(x, src3, dst3, z2, z1)
  s0 = sums[:N]
  s1 = sums[NPAD:NPAD + N]
  d0 = degs[:N].reshape(N, 1)
  d1 = degs[DPAD:DPAD + N].reshape(N, 1)
  out = _dense(s0, s1, d0, d1, x, w_l.T, w_r.T, b_l.reshape(1, D), relu)
  return out, degs


def kernel(x, edge_index, W_l1, b_l1, W_r1, W_l2, b_l2, W_r2):
  ei = edge_index.astype(jnp.int32)
  src2 = ei[0].reshape(NW, EPT)
  dst2 = ei[1].reshape(NW, EPT)
  pad_src = (jnp.arange(EPAD, dtype=jnp.int32)[None, :] * 37
             + 293 * jnp.arange(NW, dtype=jnp.int32)[:, None]) % N
  pad_dst = N + (jnp.arange(EPAD, dtype=jnp.int32)[None, :]
                 + jnp.arange(NW, dtype=jnp.int32)[:, None] * 7) % (NPAD - N)
  src3 = jnp.concatenate([src2, pad_src], axis=1).reshape(NW, NCHUNK, CHUNK)
  dst3 = jnp.concatenate([dst2, pad_dst], axis=1).reshape(NW, NCHUNK, CHUNK)
  z2 = jnp.zeros((ROWS_PER_TILE, D), jnp.float32)
  z1 = jnp.zeros((DROWS,), jnp.float32)
  h, degs = _layer(x, src3, dst3, z2, z1, W_l1, b_l1, W_r1, relu=True)
  out, _ = _layer(h, src3, dst3, z2, z1, W_l2, b_l2, W_r2, relu=False,
                  degs=degs)
  return out
